# Initial kernel scaffold; baseline (speedup 1.0000x reference)
import jax
import jax.numpy as jnp
from jax.experimental import pallas as pl


def kernel(x, w1, b1, w2, b2):
    raise NotImplementedError("write your pallas kernel here")


# speedup vs baseline: 1.3756x; 1.3756x over previous
import functools

import jax
import jax.numpy as jnp
from jax.experimental import pallas as pl
from jax.experimental.pallas import tpu as pltpu


def _se_stream_kernel(x_ref, w1t_ref, b1_ref, w2t_ref, b2_ref, o_ref,
                      acc_ref, *, nt, tt, inv_t):
    t = pl.program_id(1)

    @pl.when(t == 0)
    def _init():
        acc_ref[...] = jnp.zeros_like(acc_ref)

    x = x_ref[...]
    acc_ref[...] += jnp.sum(x.astype(jnp.float32), axis=-1)
    o_ref[:, :, pl.ds(t * tt, tt)] = x

    @pl.when(t == nt - 1)
    def _finalize():
        m = acc_ref[...] * inv_t
        h = jnp.dot(m, w1t_ref[...],
                    preferred_element_type=jnp.float32) + b1_ref[...]
        h = jnp.maximum(h, 0.0)
        y = jnp.dot(h, w2t_ref[...],
                    preferred_element_type=jnp.float32) + b2_ref[...]
        g = jax.nn.sigmoid(y).astype(o_ref.dtype)
        o_ref[...] = o_ref[...] * g[:, :, None]


def _time_tile(T):
    if T % 128 != 0:
        return T
    tt = 1024
    while T % tt != 0:
        tt //= 2
    return tt


def kernel(x, w1, b1, w2, b2):
    B, C, T = x.shape
    Cs = w1.shape[0]
    tt = _time_tile(T)
    nt = T // tt if T % tt == 0 else 1

    return pl.pallas_call(
        functools.partial(_se_stream_kernel, nt=nt, tt=tt, inv_t=1.0 / T),
        out_shape=jax.ShapeDtypeStruct((B, C, T), x.dtype),
        grid=(B, nt),
        in_specs=[
            pl.BlockSpec((1, C, tt), lambda b, t: (b, 0, t)),
            pl.BlockSpec((C, Cs), lambda b, t: (0, 0)),
            pl.BlockSpec((1, Cs), lambda b, t: (0, 0)),
            pl.BlockSpec((Cs, C), lambda b, t: (0, 0)),
            pl.BlockSpec((1, C), lambda b, t: (0, 0)),
        ],
        out_specs=pl.BlockSpec((1, C, T), lambda b, t: (b, 0, 0)),
        scratch_shapes=[pltpu.VMEM((1, C), jnp.float32)],
        compiler_params=pltpu.CompilerParams(
            dimension_semantics=("parallel", "arbitrary")),
    )(x, w1.T.astype(jnp.float32), b1.reshape(1, Cs).astype(jnp.float32),
      w2.T.astype(jnp.float32), b2.reshape(1, C).astype(jnp.float32))


# speedup vs baseline: 1.4720x; 1.0701x over previous
import functools

import jax
import jax.numpy as jnp
from jax.experimental import pallas as pl
from jax.experimental.pallas import tpu as pltpu


def _se_pipe_kernel(x_ref, w1_ref, b1_ref, w2_ref, b2_ref, o_ref,
                    ping_ref, acc_ref, g_ref, *, half, nt, tt, inv_t):
    b = pl.program_id(1)
    t = pl.program_id(2)
    cur = jax.lax.rem(b, 2)
    prv = 1 - cur

    @pl.when(b < half)
    def _stage():
        x = x_ref[0].astype(jnp.float32)
        ping_ref[cur, :, pl.ds(t * tt, tt)] = x
        part = x[:, 0:128]
        for k in range(1, tt // 128):
            part = part + x[:, k * 128:(k + 1) * 128]

        @pl.when(t == 0)
        def _():
            acc_ref[cur] = part

        @pl.when(t != 0)
        def _():
            acc_ref[cur] += part

    @pl.when(b > 0)
    def _rescale():
        @pl.when(t == 0)
        def _gate():
            m = jnp.sum(acc_ref[prv], axis=-1, keepdims=True) * inv_t
            h = jnp.dot(w1_ref[...], m,
                        preferred_element_type=jnp.float32) + b1_ref[...]
            h = jnp.maximum(h, 0.0)
            y = jnp.dot(w2_ref[...], h,
                        preferred_element_type=jnp.float32) + b2_ref[...]
            g_ref[...] = jax.nn.sigmoid(y)

        o_ref[0] = (ping_ref[prv, :, pl.ds(t * tt, tt)]
                    * g_ref[...]).astype(o_ref.dtype)


def _time_tile(T):
    if T % 128 != 0:
        return T
    tt = 1024
    while T % tt != 0:
        tt //= 2
    return tt


def kernel(x, w1, b1, w2, b2):
    B, C, T = x.shape
    Cs = w1.shape[0]
    tt = _time_tile(T)
    nt = T // tt if T % tt == 0 else 1
    nh = 2 if B % 2 == 0 else 1
    half = B // nh

    def x_idx(h, b, t):
        return (h * half + jnp.minimum(b, half - 1), 0,
                jnp.where(b == half, nt - 1, t))

    def o_idx(h, b, t):
        return (h * half + jnp.maximum(b - 1, 0), 0,
                jnp.where(b == 0, 0, t))

    const = lambda h, b, t: (0, 0)

    return pl.pallas_call(
        functools.partial(_se_pipe_kernel, half=half, nt=nt, tt=tt,
                          inv_t=1.0 / T),
        out_shape=jax.ShapeDtypeStruct((B, C, T), x.dtype),
        grid=(nh, half + 1, nt),
        in_specs=[
            pl.BlockSpec((1, C, tt), x_idx),
            pl.BlockSpec((Cs, C), const),
            pl.BlockSpec((Cs, 1), const),
            pl.BlockSpec((C, Cs), const),
            pl.BlockSpec((C, 1), const),
        ],
        out_specs=pl.BlockSpec((1, C, tt), o_idx),
        scratch_shapes=[
            pltpu.VMEM((2, C, T), jnp.float32),
            pltpu.VMEM((2, C, 128), jnp.float32),
            pltpu.VMEM((C, 1), jnp.float32),
        ],
        compiler_params=pltpu.CompilerParams(
            dimension_semantics=("parallel", "arbitrary", "arbitrary"),
            vmem_limit_bytes=48 * 1024 * 1024),
    )(x, w1.astype(jnp.float32), b1.reshape(Cs, 1).astype(jnp.float32),
      w2.astype(jnp.float32), b2.reshape(C, 1).astype(jnp.float32))


# speedup vs baseline: 1.7863x; 1.2135x over previous
import functools

import jax
import jax.numpy as jnp
from jax.experimental import pallas as pl
from jax.experimental.pallas import tpu as pltpu


def _se_pipe_kernel(x_ref, w1_ref, b1_ref, w2_ref, b2_ref, o_ref,
                    ping_ref, acc_ref, g_ref, *, half, nt, tt, inv_t):
    b = pl.program_id(1)
    t = pl.program_id(2)
    cur = jax.lax.rem(b, 2)
    prv = 1 - cur

    @pl.when(b < half)
    def _stage():
        x = x_ref[0].astype(jnp.float32)
        ping_ref[cur, :, pl.ds(t * tt, tt)] = x
        part = x[:, 0:128]
        for k in range(1, tt // 128):
            part = part + x[:, k * 128:(k + 1) * 128]

        @pl.when(t == 0)
        def _():
            acc_ref[cur] = part

        @pl.when(t != 0)
        def _():
            acc_ref[cur] += part

    @pl.when(b > 0)
    def _rescale():
        @pl.when(t == 0)
        def _gate():
            m = jnp.sum(acc_ref[prv], axis=-1, keepdims=True) * inv_t
            h = jnp.dot(w1_ref[...], m,
                        preferred_element_type=jnp.float32) + b1_ref[...]
            h = jnp.maximum(h, 0.0)
            y = jnp.dot(w2_ref[...], h,
                        preferred_element_type=jnp.float32) + b2_ref[...]
            g_ref[...] = jax.nn.sigmoid(y)

        o_ref[0] = (ping_ref[prv, :, pl.ds(t * tt, tt)]
                    * g_ref[...]).astype(o_ref.dtype)


def _time_tile(T):
    if T % 128 != 0:
        return T
    tt = 2048
    while T % tt != 0:
        tt //= 2
    return tt


def kernel(x, w1, b1, w2, b2):
    B, C, T = x.shape
    Cs = w1.shape[0]
    tt = _time_tile(T)
    nt = T // tt if T % tt == 0 else 1
    nh = 2 if B % 2 == 0 else 1
    half = B // nh

    def x_idx(h, b, t):
        return (h * half + jnp.minimum(b, half - 1), 0,
                jnp.where(b == half, nt - 1, t))

    def o_idx(h, b, t):
        return (h * half + jnp.maximum(b - 1, 0), 0,
                jnp.where(b == 0, 0, t))

    const = lambda h, b, t: (0, 0)

    return pl.pallas_call(
        functools.partial(_se_pipe_kernel, half=half, nt=nt, tt=tt,
                          inv_t=1.0 / T),
        out_shape=jax.ShapeDtypeStruct((B, C, T), x.dtype),
        grid=(nh, half + 1, nt),
        in_specs=[
            pl.BlockSpec((1, C, tt), x_idx),
            pl.BlockSpec((Cs, C), const),
            pl.BlockSpec((Cs, 1), const),
            pl.BlockSpec((C, Cs), const),
            pl.BlockSpec((C, 1), const),
        ],
        out_specs=pl.BlockSpec((1, C, tt), o_idx),
        scratch_shapes=[
            pltpu.VMEM((2, C, T), jnp.float32),
            pltpu.VMEM((2, C, 128), jnp.float32),
            pltpu.VMEM((C, 1), jnp.float32),
        ],
        compiler_params=pltpu.CompilerParams(
            dimension_semantics=("parallel", "arbitrary", "arbitrary"),
            vmem_limit_bytes=48 * 1024 * 1024),
    )(x, w1.astype(jnp.float32), b1.reshape(Cs, 1).astype(jnp.float32),
      w2.astype(jnp.float32), b2.reshape(C, 1).astype(jnp.float32))


# speedup vs baseline: 1.8840x; 1.0547x over previous
import functools

import jax
import jax.numpy as jnp
from jax.experimental import pallas as pl
from jax.experimental.pallas import tpu as pltpu


def _se_pipe_kernel(x_ref, w1_ref, b1_ref, w2_ref, b2_ref, o_ref,
                    ping_ref, acc_ref, g_ref, *, half, nt, tt, inv_t):
    b = pl.program_id(1)
    t = pl.program_id(2)
    cur = jax.lax.rem(b, 2)
    prv = 1 - cur

    @pl.when(b < half)
    def _stage():
        x = x_ref[0].astype(jnp.float32)
        ping_ref[cur, :, pl.ds(t * tt, tt)] = x
        part = x[:, 0:128]
        for k in range(1, tt // 128):
            part = part + x[:, k * 128:(k + 1) * 128]

        @pl.when(t == 0)
        def _():
            acc_ref[cur] = part

        @pl.when(t != 0)
        def _():
            acc_ref[cur] += part

    @pl.when(b > 0)
    def _rescale():
        @pl.when(t == 0)
        def _gate():
            m = jnp.sum(acc_ref[prv], axis=-1, keepdims=True) * inv_t
            h = jnp.dot(w1_ref[...], m,
                        preferred_element_type=jnp.float32) + b1_ref[...]
            h = jnp.maximum(h, 0.0)
            y = jnp.dot(w2_ref[...], h,
                        preferred_element_type=jnp.float32) + b2_ref[...]
            g_ref[...] = jax.nn.sigmoid(y)

        o_ref[0] = (ping_ref[prv, :, pl.ds(t * tt, tt)]
                    * g_ref[...]).astype(o_ref.dtype)


def _time_tile(T):
    if T % 128 != 0:
        return T
    tt = 4096
    while T % tt != 0:
        tt //= 2
    return tt


def kernel(x, w1, b1, w2, b2):
    B, C, T = x.shape
    Cs = w1.shape[0]
    tt = _time_tile(T)
    nt = T // tt if T % tt == 0 else 1
    nh = 2 if B % 2 == 0 else 1
    half = B // nh

    def x_idx(h, b, t):
        return (h * half + jnp.minimum(b, half - 1), 0,
                jnp.where(b == half, nt - 1, t))

    def o_idx(h, b, t):
        return (h * half + jnp.maximum(b - 1, 0), 0,
                jnp.where(b == 0, 0, t))

    const = lambda h, b, t: (0, 0)

    return pl.pallas_call(
        functools.partial(_se_pipe_kernel, half=half, nt=nt, tt=tt,
                          inv_t=1.0 / T),
        out_shape=jax.ShapeDtypeStruct((B, C, T), x.dtype),
        grid=(nh, half + 1, nt),
        in_specs=[
            pl.BlockSpec((1, C, tt), x_idx),
            pl.BlockSpec((Cs, C), const),
            pl.BlockSpec((Cs, 1), const),
            pl.BlockSpec((C, Cs), const),
            pl.BlockSpec((C, 1), const),
        ],
        out_specs=pl.BlockSpec((1, C, tt), o_idx),
        scratch_shapes=[
            pltpu.VMEM((2, C, T), jnp.float32),
            pltpu.VMEM((2, C, 128), jnp.float32),
            pltpu.VMEM((C, 1), jnp.float32),
        ],
        compiler_params=pltpu.CompilerParams(
            dimension_semantics=("parallel", "arbitrary", "arbitrary"),
            vmem_limit_bytes=58 * 1024 * 1024),
    )(x, w1.astype(jnp.float32), b1.reshape(Cs, 1).astype(jnp.float32),
      w2.astype(jnp.float32), b2.reshape(C, 1).astype(jnp.float32))


# speedup vs baseline: 1.9296x; 1.0242x over previous
import functools

import jax
import jax.numpy as jnp
from jax.experimental import pallas as pl
from jax.experimental.pallas import tpu as pltpu


def _se_pipe_kernel(x_ref, w1_ref, b1_ref, w2_ref, b2_ref, o_ref,
                    ping_ref, acc_ref, g_ref, *, half, nt, tt, inv_t):
    b = pl.program_id(1)
    t = pl.program_id(2)
    cur = jax.lax.rem(b, 2)
    prv = 1 - cur

    @pl.when(b < half)
    def _stage():
        x = x_ref[0].astype(jnp.float32)
        ping_ref[cur, :, pl.ds(t * tt, tt)] = x
        part = x[:, 0:128]
        for k in range(1, tt // 128):
            part = part + x[:, k * 128:(k + 1) * 128]

        @pl.when(t == 0)
        def _():
            acc_ref[cur] = part

        @pl.when(t != 0)
        def _():
            acc_ref[cur] += part

    @pl.when(b > 0)
    def _rescale():
        @pl.when(t == 0)
        def _gate():
            m = jnp.sum(acc_ref[prv], axis=-1, keepdims=True) * inv_t
            h = jnp.dot(w1_ref[...], m,
                        preferred_element_type=jnp.float32) + b1_ref[...]
            h = jnp.maximum(h, 0.0)
            y = jnp.dot(w2_ref[...], h,
                        preferred_element_type=jnp.float32) + b2_ref[...]
            g_ref[...] = jax.nn.sigmoid(y)

        o_ref[0] = (ping_ref[prv, :, pl.ds(t * tt, tt)]
                    * g_ref[...]).astype(o_ref.dtype)


def _time_tile(T):
    if T % 128 != 0:
        return T
    tt = 4096
    while T % tt != 0:
        tt //= 2
    return tt


def kernel(x, w1, b1, w2, b2):
    B, C, T = x.shape
    Cs = w1.shape[0]
    tt = _time_tile(T)
    nt = T // tt if T % tt == 0 else 1
    nh = 1
    half = B // nh

    def x_idx(h, b, t):
        return (h * half + jnp.minimum(b, half - 1), 0,
                jnp.where(b == half, nt - 1, t))

    def o_idx(h, b, t):
        return (h * half + jnp.maximum(b - 1, 0), 0,
                jnp.where(b == 0, 0, t))

    const = lambda h, b, t: (0, 0)

    return pl.pallas_call(
        functools.partial(_se_pipe_kernel, half=half, nt=nt, tt=tt,
                          inv_t=1.0 / T),
        out_shape=jax.ShapeDtypeStruct((B, C, T), x.dtype),
        grid=(nh, half + 1, nt),
        in_specs=[
            pl.BlockSpec((1, C, tt), x_idx),
            pl.BlockSpec((Cs, C), const),
            pl.BlockSpec((Cs, 1), const),
            pl.BlockSpec((C, Cs), const),
            pl.BlockSpec((C, 1), const),
        ],
        out_specs=pl.BlockSpec((1, C, tt), o_idx),
        scratch_shapes=[
            pltpu.VMEM((2, C, T), jnp.float32),
            pltpu.VMEM((2, C, 128), jnp.float32),
            pltpu.VMEM((C, 1), jnp.float32),
        ],
        compiler_params=pltpu.CompilerParams(
            dimension_semantics=("parallel", "arbitrary", "arbitrary"),
            vmem_limit_bytes=58 * 1024 * 1024),
    )(x, w1.astype(jnp.float32), b1.reshape(Cs, 1).astype(jnp.float32),
      w2.astype(jnp.float32), b2.reshape(C, 1).astype(jnp.float32))
